# baseline (speedup 1.0000x reference)
import functools

import jax
import jax.numpy as jnp
from jax import lax
from jax.experimental import pallas as pl
from jax.experimental.pallas import tpu as pltpu
from jax.experimental.pallas import tpu_sc as plsc

N = 10000
E = 320000
D = 128
DH = D // 2

NC = 2
NS = 16
NW = NC * NS
K = 128
CH = 80
EPAD = NW * CH * K
NPAD = CH * K
RPS = NPAD // NS
CHS = EPAD // (NS * K)
NBUF = 5

_mesh = plsc.VectorSubcoreMesh(core_axis_name="c", subcore_axis_name="s")


@functools.partial(
    pl.kernel,
    out_type=jax.ShapeDtypeStruct((NW, NPAD), jnp.float32),
    mesh=_mesh,
    scratch_types=[
        pltpu.VMEM((CH * K,), jnp.int32),
        pltpu.VMEM((NPAD,), jnp.float32),
    ],
    compiler_params=pltpu.CompilerParams(needs_layout_passes=False),
)
def _deg_kernel(dstf_hbm, zeros_hbm, deg_hbm, idx_v, acc_v):
    c = lax.axis_index("c")
    s = lax.axis_index("s")
    wid = s * NC + c
    pltpu.sync_copy(zeros_hbm, acc_v)
    pltpu.sync_copy(dstf_hbm.at[wid], idx_v)
    ones = jnp.full((16,), 1.0, jnp.float32)

    def body(i, carry):
        ids = idx_v[pl.ds(pl.multiple_of(i * 16, 16), 16)]
        plsc.addupdate_scatter(acc_v, [ids], ones)
        return carry

    lax.fori_loop(0, (CH * K) // 16, body, 0)
    pltpu.sync_copy(acc_v, deg_hbm.at[wid])


@functools.partial(
    pl.kernel,
    out_type=jax.ShapeDtypeStruct((NC, NPAD, DH), jnp.float32),
    mesh=_mesh,
    scratch_types=[
        pltpu.VMEM((CHS, K), jnp.int32),
        pltpu.VMEM((CHS, K), jnp.int32),
        pltpu.VMEM((NBUF, K, DH), jnp.float32),
        pltpu.VMEM_SHARED((NPAD, DH), jnp.float32),
        pltpu.SemaphoreType.DMA((NBUF,)),
        pltpu.SemaphoreType.DMA((NBUF,)),
    ],
    compiler_params=pltpu.CompilerParams(use_tc_tiling_on_sc=False),
)
def _agg_kernel(y_hbm, src_hbm, dst_hbm, parts_hbm, src_v, dst_v, rows_v, acc_sh, gsem, ssem):
    c = lax.axis_index("c")
    s = lax.axis_index("s")
    yc = y_hbm.at[c]
    row0 = pl.multiple_of(s * RPS, 8)
    pltpu.sync_copy(yc.at[pl.ds(row0, RPS)], acc_sh.at[pl.ds(row0, RPS)])
    pltpu.sync_copy(src_hbm.at[s], src_v)
    pltpu.sync_copy(dst_hbm.at[s], dst_v)
    plsc.subcore_barrier()

    for b in range(NBUF - 1):
        pltpu.async_copy(yc.at[src_v.at[b]], rows_v.at[b], gsem.at[b])

    def step(j, carry):
        b = lax.rem(j, NBUF)
        pltpu.make_async_copy(yc.at[src_v.at[0]], rows_v.at[b], gsem.at[b]).wait()
        pltpu.async_copy(rows_v.at[b], acc_sh.at[dst_v.at[j]], ssem.at[b], add=True)
        jg = j + NBUF - 1
        bg = lax.rem(jg, NBUF)

        @pl.when(jg < CHS)
        def _fire():
            @pl.when(j >= 1)
            def _drain():
                pltpu.make_async_copy(
                    rows_v.at[bg], acc_sh.at[dst_v.at[0]], ssem.at[bg]).wait()
            pltpu.async_copy(yc.at[src_v.at[jg]], rows_v.at[bg], gsem.at[bg])
        return carry

    lax.fori_loop(0, CHS, step, 0)
    for b in range(NBUF):
        pltpu.make_async_copy(rows_v.at[b], acc_sh.at[dst_v.at[0]], ssem.at[b]).wait()
    plsc.subcore_barrier()
    pltpu.sync_copy(acc_sh.at[pl.ds(row0, RPS)], parts_hbm.at[c, pl.ds(row0, RPS)])


_BLK = 128


def _tc1_body(parts_ref, x_ref, w_ref, y_ref, dis_ref):
    deg = jnp.sum(parts_ref[...], axis=0) + 1.0
    dis = lax.rsqrt(deg)
    y = jnp.dot(x_ref[...], w_ref[...], preferred_element_type=jnp.float32)
    y = y * dis[:, None]
    y_ref[0] = y[:, :DH]
    y_ref[1] = y[:, DH:]
    dis_ref[...] = dis


def _tc1(deg_parts, x_pad, W1):
    return pl.pallas_call(
        _tc1_body,
        grid=(NPAD // _BLK,),
        in_specs=[
            pl.BlockSpec((NW, _BLK), lambda i: (0, i)),
            pl.BlockSpec((_BLK, D), lambda i: (i, 0)),
            pl.BlockSpec((D, D), lambda i: (0, 0)),
        ],
        out_specs=[
            pl.BlockSpec((NC, _BLK, DH), lambda i: (0, i, 0)),
            pl.BlockSpec((_BLK,), lambda i: (i,)),
        ],
        out_shape=[
            jax.ShapeDtypeStruct((NC, NPAD, DH), jnp.float32),
            jax.ShapeDtypeStruct((NPAD,), jnp.float32),
        ],
    )(deg_parts, x_pad, W1)


def _tc2_body(parts_ref, dis_ref, b_ref, w_ref, y2_ref):
    agg = jnp.concatenate([parts_ref[0], parts_ref[1]], axis=-1)
    dis = dis_ref[...]
    h = jnp.maximum(agg * dis[:, None] + b_ref[...][None, :], 0.0)
    y2 = jnp.dot(h, w_ref[...], preferred_element_type=jnp.float32)
    y2 = y2 * dis[:, None]
    y2_ref[0] = y2[:, :DH]
    y2_ref[1] = y2[:, DH:]


def _tc2(parts, dis, b1, W2):
    return pl.pallas_call(
        _tc2_body,
        grid=(NPAD // _BLK,),
        in_specs=[
            pl.BlockSpec((NC, _BLK, DH), lambda i: (0, i, 0)),
            pl.BlockSpec((_BLK,), lambda i: (i,)),
            pl.BlockSpec((D,), lambda i: (0,)),
            pl.BlockSpec((D, D), lambda i: (0, 0)),
        ],
        out_specs=pl.BlockSpec((NC, _BLK, DH), lambda i: (0, i, 0)),
        out_shape=jax.ShapeDtypeStruct((NC, NPAD, DH), jnp.float32),
    )(parts, dis, b1, W2)


def _tc3_body(parts_ref, dis_ref, b_ref, o_ref):
    agg = jnp.concatenate([parts_ref[0], parts_ref[1]], axis=-1)
    o_ref[...] = jnp.maximum(
        agg * dis_ref[...][:, None] + b_ref[...][None, :], 0.0)


def _tc3(parts, dis, b2):
    return pl.pallas_call(
        _tc3_body,
        grid=(NPAD // _BLK,),
        in_specs=[
            pl.BlockSpec((NC, _BLK, DH), lambda i: (0, i, 0)),
            pl.BlockSpec((_BLK,), lambda i: (i,)),
            pl.BlockSpec((D,), lambda i: (0,)),
        ],
        out_specs=pl.BlockSpec((_BLK, D), lambda i: (i, 0)),
        out_shape=jax.ShapeDtypeStruct((NPAD, D), jnp.float32),
    )(parts, dis, b2)


def kernel(x, edge_index, W1, b1, W2, b2):
    src = edge_index[0]
    dst = edge_index[1]
    pad_e = EPAD - E
    src_p = jnp.concatenate([src, jnp.zeros((pad_e,), jnp.int32)])
    dst_p = jnp.concatenate([dst, jnp.full((pad_e,), N, jnp.int32)])
    src3 = src_p.reshape(NS, CHS, K)
    dst3 = dst_p.reshape(NS, CHS, K)
    dst_flat = dst_p.reshape(NW, CH * K)
    x_pad = jnp.pad(x, ((0, NPAD - N), (0, 0)))
    zeros_row = jnp.zeros((NPAD,), jnp.float32)

    deg_parts = _deg_kernel(dst_flat, zeros_row)
    y1, dis = _tc1(deg_parts, x_pad, W1)
    parts1 = _agg_kernel(y1, src3, dst3)
    y2 = _tc2(parts1, dis, b1, W2)
    parts2 = _agg_kernel(y2, src3, dst3)
    out = _tc3(parts2, dis, b2)
    return out[:N]


# speedup vs baseline: 1.8273x; 1.8273x over previous
import functools

import jax
import jax.numpy as jnp
from jax import lax
from jax.experimental import pallas as pl
from jax.experimental.pallas import tpu as pltpu
from jax.experimental.pallas import tpu_sc as plsc

N = 10000
E = 320000
D = 128
DH = D // 2

NC = 2
NS = 16
NW = NC * NS
K = 128
CH = 80
EPAD = NW * CH * K
NPAD = CH * K
RPS = NPAD // NS
CHS = EPAD // (NS * K)
NBUF = 5

_mesh = plsc.VectorSubcoreMesh(core_axis_name="c", subcore_axis_name="s")


@functools.partial(
    pl.kernel,
    out_type=jax.ShapeDtypeStruct((NW, NPAD), jnp.float32),
    mesh=_mesh,
    scratch_types=[
        pltpu.VMEM((CH * K,), jnp.int32),
        pltpu.VMEM((NPAD,), jnp.float32),
    ],
    compiler_params=pltpu.CompilerParams(needs_layout_passes=False),
)
def _deg_kernel(dstf_hbm, zeros_hbm, deg_hbm, idx_v, acc_v):
    c = lax.axis_index("c")
    s = lax.axis_index("s")
    wid = s * NC + c
    pltpu.sync_copy(zeros_hbm, acc_v)
    pltpu.sync_copy(dstf_hbm.at[wid], idx_v)
    ones = jnp.full((16,), 1.0, jnp.float32)

    def body(i, carry):
        ids = idx_v[pl.ds(pl.multiple_of(i * 16, 16), 16)]
        plsc.addupdate_scatter(acc_v, [ids], ones)
        return carry

    lax.fori_loop(0, (CH * K) // 16, body, 0)
    pltpu.sync_copy(acc_v, deg_hbm.at[wid])


IB = 10
GL = 2
IL = 4


@functools.partial(
    pl.kernel,
    out_type=jax.ShapeDtypeStruct((NC, NPAD, DH), jnp.float32),
    mesh=_mesh,
    scratch_types=[
        pltpu.VMEM((IB, K), jnp.int32),
        pltpu.VMEM((IB, K), jnp.int32),
        pltpu.VMEM((NBUF, K, DH), jnp.float32),
        pltpu.VMEM_SHARED((NPAD, DH), jnp.float32),
        pltpu.VMEM_SHARED((NPAD, DH), jnp.float32),
        pltpu.SemaphoreType.DMA((IB,)),
        pltpu.SemaphoreType.DMA((NBUF,)),
        pltpu.SemaphoreType.DMA((NBUF,)),
    ],
    compiler_params=pltpu.CompilerParams(use_tc_tiling_on_sc=False),
)
def _agg_kernel(y_hbm, src_hbm, dst_hbm, parts_hbm,
                sidx_v, didx_v, rows_v, y_sh, acc_sh, isem, gsem, ssem):
    c = lax.axis_index("c")
    s = lax.axis_index("s")
    yc = y_hbm.at[c]
    row0 = pl.multiple_of(s * RPS, 8)
    pltpu.sync_copy(yc.at[pl.ds(row0, RPS)], y_sh.at[pl.ds(row0, RPS)])
    pltpu.sync_copy(yc.at[pl.ds(row0, RPS)], acc_sh.at[pl.ds(row0, RPS)])
    plsc.subcore_barrier()

    def fire_idx(jj):
        bi = lax.rem(jj, IB)
        pltpu.async_copy(src_hbm.at[s, jj], sidx_v.at[bi], isem.at[bi])
        pltpu.async_copy(dst_hbm.at[s, jj], didx_v.at[bi], isem.at[bi])

    def wait_idx(jj):
        bi = lax.rem(jj, IB)
        pltpu.make_async_copy(src_hbm.at[s, 0], sidx_v.at[bi], isem.at[bi]).wait()
        pltpu.make_async_copy(src_hbm.at[s, 0], didx_v.at[bi], isem.at[bi]).wait()

    def fire_gather(jj):
        bi = lax.rem(jj, IB)
        bg = lax.rem(jj, NBUF)
        pltpu.async_copy(y_sh.at[sidx_v.at[bi]], rows_v.at[bg], gsem.at[bg])

    for jj in range(IL):
        fire_idx(jj)
    for jj in range(GL):
        wait_idx(jj)
        fire_gather(jj)

    def step(j, carry):
        b = lax.rem(j, NBUF)
        pltpu.make_async_copy(y_sh.at[sidx_v.at[0]], rows_v.at[b], gsem.at[b]).wait()
        pltpu.async_copy(rows_v.at[b], acc_sh.at[didx_v.at[lax.rem(j, IB)]],
                         ssem.at[b], add=True)

        @pl.when(j + IL < CHS)
        def _idx():
            fire_idx(j + IL)

        @pl.when(j + GL < CHS)
        def _gather():
            @pl.when(j + GL - NBUF >= 0)
            def _drain():
                bq = lax.rem(j + GL, NBUF)
                pltpu.make_async_copy(
                    rows_v.at[bq], acc_sh.at[didx_v.at[0]], ssem.at[bq]).wait()
            wait_idx(j + GL)
            fire_gather(j + GL)
        return carry

    lax.fori_loop(0, CHS, step, 0)
    for j in range(CHS - NBUF, CHS):
        b = j % NBUF
        pltpu.make_async_copy(rows_v.at[b], acc_sh.at[didx_v.at[0]], ssem.at[b]).wait()
    plsc.subcore_barrier()
    pltpu.sync_copy(acc_sh.at[pl.ds(row0, RPS)], parts_hbm.at[c, pl.ds(row0, RPS)])


_BLK = 128


def _tc1_body(parts_ref, x_ref, w_ref, y_ref, dis_ref):
    deg = jnp.sum(parts_ref[...], axis=0) + 1.0
    dis = lax.rsqrt(deg)
    y = jnp.dot(x_ref[...], w_ref[...], preferred_element_type=jnp.float32)
    y = y * dis[:, None]
    y_ref[0] = y[:, :DH]
    y_ref[1] = y[:, DH:]
    dis_ref[...] = dis


def _tc1(deg_parts, x_pad, W1):
    return pl.pallas_call(
        _tc1_body,
        grid=(NPAD // _BLK,),
        in_specs=[
            pl.BlockSpec((NW, _BLK), lambda i: (0, i)),
            pl.BlockSpec((_BLK, D), lambda i: (i, 0)),
            pl.BlockSpec((D, D), lambda i: (0, 0)),
        ],
        out_specs=[
            pl.BlockSpec((NC, _BLK, DH), lambda i: (0, i, 0)),
            pl.BlockSpec((_BLK,), lambda i: (i,)),
        ],
        out_shape=[
            jax.ShapeDtypeStruct((NC, NPAD, DH), jnp.float32),
            jax.ShapeDtypeStruct((NPAD,), jnp.float32),
        ],
    )(deg_parts, x_pad, W1)


def _tc2_body(parts_ref, dis_ref, b_ref, w_ref, y2_ref):
    agg = jnp.concatenate([parts_ref[0], parts_ref[1]], axis=-1)
    dis = dis_ref[...]
    h = jnp.maximum(agg * dis[:, None] + b_ref[...][None, :], 0.0)
    y2 = jnp.dot(h, w_ref[...], preferred_element_type=jnp.float32)
    y2 = y2 * dis[:, None]
    y2_ref[0] = y2[:, :DH]
    y2_ref[1] = y2[:, DH:]


def _tc2(parts, dis, b1, W2):
    return pl.pallas_call(
        _tc2_body,
        grid=(NPAD // _BLK,),
        in_specs=[
            pl.BlockSpec((NC, _BLK, DH), lambda i: (0, i, 0)),
            pl.BlockSpec((_BLK,), lambda i: (i,)),
            pl.BlockSpec((D,), lambda i: (0,)),
            pl.BlockSpec((D, D), lambda i: (0, 0)),
        ],
        out_specs=pl.BlockSpec((NC, _BLK, DH), lambda i: (0, i, 0)),
        out_shape=jax.ShapeDtypeStruct((NC, NPAD, DH), jnp.float32),
    )(parts, dis, b1, W2)


def _tc3_body(parts_ref, dis_ref, b_ref, o_ref):
    agg = jnp.concatenate([parts_ref[0], parts_ref[1]], axis=-1)
    o_ref[...] = jnp.maximum(
        agg * dis_ref[...][:, None] + b_ref[...][None, :], 0.0)


def _tc3(parts, dis, b2):
    return pl.pallas_call(
        _tc3_body,
        grid=(NPAD // _BLK,),
        in_specs=[
            pl.BlockSpec((NC, _BLK, DH), lambda i: (0, i, 0)),
            pl.BlockSpec((_BLK,), lambda i: (i,)),
            pl.BlockSpec((D,), lambda i: (0,)),
        ],
        out_specs=pl.BlockSpec((_BLK, D), lambda i: (i, 0)),
        out_shape=jax.ShapeDtypeStruct((NPAD, D), jnp.float32),
    )(parts, dis, b2)


def kernel(x, edge_index, W1, b1, W2, b2):
    src = edge_index[0]
    dst = edge_index[1]
    pad_e = EPAD - E
    src_p = jnp.concatenate([src, jnp.zeros((pad_e,), jnp.int32)])
    dst_p = jnp.concatenate([dst, jnp.full((pad_e,), N, jnp.int32)])
    src3 = src_p.reshape(NS, CHS, K)
    dst3 = dst_p.reshape(NS, CHS, K)
    dst_flat = dst_p.reshape(NW, CH * K)
    x_pad = jnp.pad(x, ((0, NPAD - N), (0, 0)))
    zeros_row = jnp.zeros((NPAD,), jnp.float32)

    deg_parts = _deg_kernel(dst_flat, zeros_row)
    y1, dis = _tc1(deg_parts, x_pad, W1)
    parts1 = _agg_kernel(y1, src3, dst3)
    y2 = _tc2(parts1, dis, b1, W2)
    parts2 = _agg_kernel(y2, src3, dst3)
    out = _tc3(parts2, dis, b2)
    return out[:N]


# speedup vs baseline: 2.8654x; 1.5681x over previous
import functools

import jax
import jax.numpy as jnp
from jax import lax
from jax.experimental import pallas as pl
from jax.experimental.pallas import tpu as pltpu
from jax.experimental.pallas import tpu_sc as plsc

N = 10000
E = 320000
D = 128
DH = D // 2

NC = 2
NS = 16
NW = NC * NS
K = 128
NPAD = 10240
RPS = NPAD // NS
ESUB = E // NS
CHS = -(-ESUB // K)
TAIL = ESUB - (CHS - 1) * K
NBUF = 5

_mesh = plsc.VectorSubcoreMesh(core_axis_name="c", subcore_axis_name="s")


EW = E // NW


@functools.partial(
    pl.kernel,
    out_type=jax.ShapeDtypeStruct((NW, NPAD), jnp.float32),
    mesh=_mesh,
    scratch_types=[
        pltpu.VMEM((EW,), jnp.int32),
        pltpu.VMEM((NPAD,), jnp.float32),
    ],
    compiler_params=pltpu.CompilerParams(needs_layout_passes=False),
)
def _deg_kernel(dst_hbm, zeros_hbm, deg_hbm, idx_v, acc_v):
    c = lax.axis_index("c")
    s = lax.axis_index("s")
    wid = s * NC + c
    pltpu.sync_copy(zeros_hbm, acc_v)
    pltpu.sync_copy(dst_hbm.at[pl.ds(wid * EW, EW)], idx_v)
    ones = jnp.full((16,), 1.0, jnp.float32)

    def body(i, carry):
        ids = idx_v[pl.ds(pl.multiple_of(i * 16, 16), 16)]
        plsc.addupdate_scatter(acc_v, [ids], ones)
        return carry

    lax.fori_loop(0, EW // 16, body, 0)
    pltpu.sync_copy(acc_v, deg_hbm.at[wid])


IB = 10
GL = 2
IL = 4


@functools.partial(
    pl.kernel,
    out_type=jax.ShapeDtypeStruct((NPAD, D), jnp.float32),
    mesh=_mesh,
    scratch_types=[
        pltpu.VMEM((IB, K), jnp.int32),
        pltpu.VMEM((IB, K), jnp.int32),
        pltpu.VMEM((NBUF, K, DH), jnp.float32),
        pltpu.VMEM_SHARED((NPAD, DH), jnp.float32),
        pltpu.VMEM_SHARED((NPAD, DH), jnp.float32),
        pltpu.SemaphoreType.DMA((IB,)),
        pltpu.SemaphoreType.DMA((NBUF,)),
        pltpu.SemaphoreType.DMA((NBUF,)),
    ],
    compiler_params=pltpu.CompilerParams(use_tc_tiling_on_sc=False),
)
def _agg_kernel(y_hbm, src_hbm, dst_hbm, safe_src, safe_dst, parts_hbm,
                sidx_v, didx_v, rows_v, y_sh, acc_sh, isem, gsem, ssem):
    c = lax.axis_index("c")
    s = lax.axis_index("s")
    col0 = pl.multiple_of(c * DH, 8)
    row0 = pl.multiple_of(s * RPS, 8)
    ebase = pl.multiple_of(s * ESUB, 8)
    pltpu.sync_copy(y_hbm.at[pl.ds(row0, RPS), pl.ds(col0, DH)],
                    y_sh.at[pl.ds(row0, RPS)])
    pltpu.sync_copy(y_hbm.at[pl.ds(row0, RPS), pl.ds(col0, DH)],
                    acc_sh.at[pl.ds(row0, RPS)])
    plsc.subcore_barrier()

    def fire_idx(jj):
        bi = jj % IB if isinstance(jj, int) else lax.rem(jj, IB)
        off = ebase + jj * K
        if isinstance(jj, int):
            pltpu.async_copy(src_hbm.at[pl.ds(off, K)], sidx_v.at[bi], isem.at[bi])
            pltpu.async_copy(dst_hbm.at[pl.ds(off, K)], didx_v.at[bi], isem.at[bi])
            return

        @pl.when(jj == CHS - 1)
        def _tail():
            pltpu.sync_copy(safe_src, sidx_v.at[bi])
            pltpu.sync_copy(safe_dst, didx_v.at[bi])
            pltpu.async_copy(src_hbm.at[pl.ds(off, TAIL)],
                             sidx_v.at[bi, pl.ds(0, TAIL)], isem.at[bi])
            pltpu.async_copy(dst_hbm.at[pl.ds(off, TAIL)],
                             didx_v.at[bi, pl.ds(0, TAIL)], isem.at[bi])

        @pl.when(jj != CHS - 1)
        def _full():
            pltpu.async_copy(src_hbm.at[pl.ds(off, K)], sidx_v.at[bi], isem.at[bi])
            pltpu.async_copy(dst_hbm.at[pl.ds(off, K)], didx_v.at[bi], isem.at[bi])

    def wait_idx(jj):
        bi = jj % IB if isinstance(jj, int) else lax.rem(jj, IB)
        if isinstance(jj, int):
            pltpu.make_async_copy(src_hbm.at[pl.ds(0, K)], sidx_v.at[bi], isem.at[bi]).wait()
            pltpu.make_async_copy(src_hbm.at[pl.ds(0, K)], didx_v.at[bi], isem.at[bi]).wait()
            return

        @pl.when(jj == CHS - 1)
        def _tail():
            pltpu.make_async_copy(src_hbm.at[pl.ds(0, TAIL)],
                                  sidx_v.at[bi, pl.ds(0, TAIL)], isem.at[bi]).wait()
            pltpu.make_async_copy(src_hbm.at[pl.ds(0, TAIL)],
                                  didx_v.at[bi, pl.ds(0, TAIL)], isem.at[bi]).wait()

        @pl.when(jj != CHS - 1)
        def _full():
            pltpu.make_async_copy(src_hbm.at[pl.ds(0, K)], sidx_v.at[bi], isem.at[bi]).wait()
            pltpu.make_async_copy(src_hbm.at[pl.ds(0, K)], didx_v.at[bi], isem.at[bi]).wait()

    def fire_gather(jj):
        bi = lax.rem(jj, IB)
        bg = lax.rem(jj, NBUF)
        pltpu.async_copy(y_sh.at[sidx_v.at[bi]], rows_v.at[bg], gsem.at[bg])

    for jj in range(IL):
        fire_idx(jj)
    for jj in range(GL):
        wait_idx(jj)
        fire_gather(jj)

    def step(j, carry):
        b = lax.rem(j, NBUF)
        pltpu.make_async_copy(y_sh.at[sidx_v.at[0]], rows_v.at[b], gsem.at[b]).wait()
        pltpu.async_copy(rows_v.at[b], acc_sh.at[didx_v.at[lax.rem(j, IB)]],
                         ssem.at[b], add=True)

        @pl.when(j + IL < CHS)
        def _idx():
            fire_idx(j + IL)

        @pl.when(j + GL < CHS)
        def _gather():
            @pl.when(j + GL - NBUF >= 0)
            def _drain():
                bq = lax.rem(j + GL, NBUF)
                pltpu.make_async_copy(
                    rows_v.at[bq], acc_sh.at[didx_v.at[0]], ssem.at[bq]).wait()
            wait_idx(j + GL)
            fire_gather(j + GL)
        return carry

    lax.fori_loop(0, CHS, step, 0)
    for j in range(CHS - NBUF, CHS):
        b = j % NBUF
        pltpu.make_async_copy(rows_v.at[b], acc_sh.at[didx_v.at[0]], ssem.at[b]).wait()
    plsc.subcore_barrier()
    pltpu.sync_copy(acc_sh.at[pl.ds(row0, RPS)],
                    parts_hbm.at[pl.ds(row0, RPS), pl.ds(col0, DH)])


_BLK = 1280


def _tc1_body(parts_ref, x_ref, w_ref, y_ref, dis_ref):
    deg = jnp.sum(parts_ref[...], axis=0) + 1.0
    dis = lax.rsqrt(deg)
    y = jnp.dot(x_ref[...], w_ref[...], preferred_element_type=jnp.float32)
    y_ref[...] = y * dis[:, None]
    dis_ref[...] = dis[:, None]


def _tc1(deg_parts, x_pad, W1):
    return pl.pallas_call(
        _tc1_body,
        grid=(NPAD // _BLK,),
        in_specs=[
            pl.BlockSpec((NW, _BLK), lambda i: (0, i)),
            pl.BlockSpec((_BLK, D), lambda i: (i, 0)),
            pl.BlockSpec((D, D), lambda i: (0, 0)),
        ],
        out_specs=[
            pl.BlockSpec((_BLK, D), lambda i: (i, 0)),
            pl.BlockSpec((_BLK, 1), lambda i: (i, 0)),
        ],
        out_shape=[
            jax.ShapeDtypeStruct((NPAD, D), jnp.float32),
            jax.ShapeDtypeStruct((NPAD, 1), jnp.float32),
        ],
    )(deg_parts, x_pad, W1)


def _tc2_body(agg_ref, dis_ref, b_ref, w_ref, y2_ref):
    dis = dis_ref[...]
    h = jnp.maximum(agg_ref[...] * dis + b_ref[...][None, :], 0.0)
    y2 = jnp.dot(h, w_ref[...], preferred_element_type=jnp.float32)
    y2_ref[...] = y2 * dis


def _tc2(agg, dis, b1, W2):
    return pl.pallas_call(
        _tc2_body,
        grid=(NPAD // _BLK,),
        in_specs=[
            pl.BlockSpec((_BLK, D), lambda i: (i, 0)),
            pl.BlockSpec((_BLK, 1), lambda i: (i, 0)),
            pl.BlockSpec((D,), lambda i: (0,)),
            pl.BlockSpec((D, D), lambda i: (0, 0)),
        ],
        out_specs=pl.BlockSpec((_BLK, D), lambda i: (i, 0)),
        out_shape=jax.ShapeDtypeStruct((NPAD, D), jnp.float32),
    )(agg, dis, b1, W2)


def _tc3_body(agg_ref, dis_ref, b_ref, o_ref):
    o_ref[...] = jnp.maximum(
        agg_ref[...] * dis_ref[...] + b_ref[...][None, :], 0.0)


def _tc3(agg, dis, b2):
    return pl.pallas_call(
        _tc3_body,
        grid=(NPAD // _BLK,),
        in_specs=[
            pl.BlockSpec((_BLK, D), lambda i: (i, 0)),
            pl.BlockSpec((_BLK, 1), lambda i: (i, 0)),
            pl.BlockSpec((D,), lambda i: (0,)),
        ],
        out_specs=pl.BlockSpec((_BLK, D), lambda i: (i, 0)),
        out_shape=jax.ShapeDtypeStruct((N, D), jnp.float32),
    )(agg, dis, b2)


def kernel(x, edge_index, W1, b1, W2, b2):
    src = edge_index[0]
    dst = edge_index[1]
    zeros_row = jnp.zeros((NPAD,), jnp.float32)
    safe_src = jnp.zeros((K,), jnp.int32)
    safe_dst = jnp.full((K,), N, jnp.int32)

    deg_parts = _deg_kernel(dst, zeros_row)
    y1, dis = _tc1(deg_parts, x, W1)
    agg1 = _agg_kernel(y1, src, dst, safe_src, safe_dst)
    y2 = _tc2(agg1, dis, b1, W2)
    agg2 = _agg_kernel(y2, src, dst, safe_src, safe_dst)
    return _tc3(agg2, dis, b2)


# speedup vs baseline: 2.9726x; 1.0374x over previous
import functools

import jax
import jax.numpy as jnp
from jax import lax
from jax.experimental import pallas as pl
from jax.experimental.pallas import tpu as pltpu
from jax.experimental.pallas import tpu_sc as plsc

N = 10000
E = 320000
D = 128
DH = D // 2

NC = 2
NS = 16
NW = NC * NS
K = 128
NPAD = 10240
RPS = NPAD // NS
ESUB = E // NS
CHS = -(-ESUB // K)
TAIL = ESUB - (CHS - 1) * K
NBUF = 5

_mesh = plsc.VectorSubcoreMesh(core_axis_name="c", subcore_axis_name="s")


EW = E // NW


@functools.partial(
    pl.kernel,
    out_type=jax.ShapeDtypeStruct((NW, NPAD), jnp.float32),
    mesh=_mesh,
    scratch_types=[
        pltpu.VMEM((EW,), jnp.int32),
        pltpu.VMEM((NPAD,), jnp.float32),
    ],
    compiler_params=pltpu.CompilerParams(needs_layout_passes=False),
)
def _deg_kernel(edges_hbm, zeros_hbm, deg_hbm, idx_v, acc_v):
    c = lax.axis_index("c")
    s = lax.axis_index("s")
    wid = s * NC + c
    pltpu.sync_copy(zeros_hbm, acc_v)
    pltpu.sync_copy(edges_hbm.at[pl.ds(E + wid * EW, EW)], idx_v)
    ones = jnp.full((16,), 1.0, jnp.float32)

    def body(i, carry):
        ids = idx_v[pl.ds(pl.multiple_of(i * 16, 16), 16)]
        plsc.addupdate_scatter(acc_v, [ids], ones)
        return carry

    lax.fori_loop(0, EW // 16, body, 0)
    pltpu.sync_copy(acc_v, deg_hbm.at[wid])


IB = 10
GL = 2
IL = 4


@functools.partial(
    pl.kernel,
    out_type=jax.ShapeDtypeStruct((NPAD, D), jnp.float32),
    mesh=_mesh,
    scratch_types=[
        pltpu.VMEM((IB, K), jnp.int32),
        pltpu.VMEM((IB, K), jnp.int32),
        pltpu.VMEM((NBUF, K, DH), jnp.float32),
        pltpu.VMEM_SHARED((NPAD, DH), jnp.float32),
        pltpu.VMEM_SHARED((NPAD, DH), jnp.float32),
        pltpu.SemaphoreType.DMA((IB,)),
        pltpu.SemaphoreType.DMA((NBUF,)),
        pltpu.SemaphoreType.DMA((NBUF,)),
    ],
    compiler_params=pltpu.CompilerParams(use_tc_tiling_on_sc=False),
)
def _agg_kernel(y_hbm, edges_hbm, safe_src, safe_dst, parts_hbm,
                sidx_v, didx_v, rows_v, y_sh, acc_sh, isem, gsem, ssem):
    c = lax.axis_index("c")
    s = lax.axis_index("s")
    col0 = pl.multiple_of(c * DH, 8)
    row0 = pl.multiple_of(s * RPS, 8)
    ebase = pl.multiple_of(s * ESUB, 8)
    pltpu.sync_copy(y_hbm.at[pl.ds(row0, RPS), pl.ds(col0, DH)],
                    y_sh.at[pl.ds(row0, RPS)])
    pltpu.sync_copy(y_hbm.at[pl.ds(row0, RPS), pl.ds(col0, DH)],
                    acc_sh.at[pl.ds(row0, RPS)])
    plsc.subcore_barrier()

    def fire_idx(jj):
        bi = jj % IB if isinstance(jj, int) else lax.rem(jj, IB)
        soff = ebase + jj * K
        doff = E + ebase + jj * K
        if isinstance(jj, int):
            pltpu.async_copy(edges_hbm.at[pl.ds(soff, K)], sidx_v.at[bi], isem.at[bi])
            pltpu.async_copy(edges_hbm.at[pl.ds(doff, K)], didx_v.at[bi], isem.at[bi])
            return

        @pl.when(jj == CHS - 1)
        def _tail():
            pltpu.sync_copy(safe_src, sidx_v.at[bi])
            pltpu.sync_copy(safe_dst, didx_v.at[bi])
            pltpu.async_copy(edges_hbm.at[pl.ds(soff, TAIL)],
                             sidx_v.at[bi, pl.ds(0, TAIL)], isem.at[bi])
            pltpu.async_copy(edges_hbm.at[pl.ds(doff, TAIL)],
                             didx_v.at[bi, pl.ds(0, TAIL)], isem.at[bi])

        @pl.when(jj != CHS - 1)
        def _full():
            pltpu.async_copy(edges_hbm.at[pl.ds(soff, K)], sidx_v.at[bi], isem.at[bi])
            pltpu.async_copy(edges_hbm.at[pl.ds(doff, K)], didx_v.at[bi], isem.at[bi])

    def wait_idx(jj):
        bi = jj % IB if isinstance(jj, int) else lax.rem(jj, IB)
        if isinstance(jj, int):
            pltpu.make_async_copy(edges_hbm.at[pl.ds(0, K)], sidx_v.at[bi], isem.at[bi]).wait()
            pltpu.make_async_copy(edges_hbm.at[pl.ds(0, K)], didx_v.at[bi], isem.at[bi]).wait()
            return

        @pl.when(jj == CHS - 1)
        def _tail():
            pltpu.make_async_copy(edges_hbm.at[pl.ds(0, TAIL)],
                                  sidx_v.at[bi, pl.ds(0, TAIL)], isem.at[bi]).wait()
            pltpu.make_async_copy(edges_hbm.at[pl.ds(0, TAIL)],
                                  didx_v.at[bi, pl.ds(0, TAIL)], isem.at[bi]).wait()

        @pl.when(jj != CHS - 1)
        def _full():
            pltpu.make_async_copy(edges_hbm.at[pl.ds(0, K)], sidx_v.at[bi], isem.at[bi]).wait()
            pltpu.make_async_copy(edges_hbm.at[pl.ds(0, K)], didx_v.at[bi], isem.at[bi]).wait()

    def fire_gather(jj):
        bi = lax.rem(jj, IB)
        bg = lax.rem(jj, NBUF)
        pltpu.async_copy(y_sh.at[sidx_v.at[bi]], rows_v.at[bg], gsem.at[bg])

    for jj in range(IL):
        fire_idx(jj)
    for jj in range(GL):
        wait_idx(jj)
        fire_gather(jj)

    def step(j, carry):
        b = lax.rem(j, NBUF)
        pltpu.make_async_copy(y_sh.at[sidx_v.at[0]], rows_v.at[b], gsem.at[b]).wait()
        pltpu.async_copy(rows_v.at[b], acc_sh.at[didx_v.at[lax.rem(j, IB)]],
                         ssem.at[b], add=True)

        @pl.when(j + IL < CHS)
        def _idx():
            fire_idx(j + IL)

        @pl.when(j + GL < CHS)
        def _gather():
            @pl.when(j + GL - NBUF >= 0)
            def _drain():
                bq = lax.rem(j + GL, NBUF)
                pltpu.make_async_copy(
                    rows_v.at[bq], acc_sh.at[didx_v.at[0]], ssem.at[bq]).wait()
            wait_idx(j + GL)
            fire_gather(j + GL)
        return carry

    lax.fori_loop(0, CHS, step, 0)
    for j in range(CHS - NBUF, CHS):
        b = j % NBUF
        pltpu.make_async_copy(rows_v.at[b], acc_sh.at[didx_v.at[0]], ssem.at[b]).wait()
    plsc.subcore_barrier()
    pltpu.sync_copy(acc_sh.at[pl.ds(row0, RPS)],
                    parts_hbm.at[pl.ds(row0, RPS), pl.ds(col0, DH)])


_BLK = 1280


def _tc1_body(parts_ref, x_ref, w_ref, y_ref, dis_ref):
    deg = jnp.sum(parts_ref[...], axis=0) + 1.0
    dis = lax.rsqrt(deg)
    y = jnp.dot(x_ref[...], w_ref[...], preferred_element_type=jnp.float32)
    y_ref[...] = y * dis[:, None]
    dis_ref[...] = dis[:, None]


def _tc1(deg_parts, x_pad, W1):
    return pl.pallas_call(
        _tc1_body,
        grid=(NPAD // _BLK,),
        in_specs=[
            pl.BlockSpec((NW, _BLK), lambda i: (0, i)),
            pl.BlockSpec((_BLK, D), lambda i: (i, 0)),
            pl.BlockSpec((D, D), lambda i: (0, 0)),
        ],
        out_specs=[
            pl.BlockSpec((_BLK, D), lambda i: (i, 0)),
            pl.BlockSpec((_BLK, 1), lambda i: (i, 0)),
        ],
        out_shape=[
            jax.ShapeDtypeStruct((NPAD, D), jnp.float32),
            jax.ShapeDtypeStruct((NPAD, 1), jnp.float32),
        ],
    )(deg_parts, x_pad, W1)


def _tc2_body(agg_ref, dis_ref, b_ref, w_ref, y2_ref):
    dis = dis_ref[...]
    h = jnp.maximum(agg_ref[...] * dis + b_ref[...][None, :], 0.0)
    y2 = jnp.dot(h, w_ref[...], preferred_element_type=jnp.float32)
    y2_ref[...] = y2 * dis


def _tc2(agg, dis, b1, W2):
    return pl.pallas_call(
        _tc2_body,
        grid=(NPAD // _BLK,),
        in_specs=[
            pl.BlockSpec((_BLK, D), lambda i: (i, 0)),
            pl.BlockSpec((_BLK, 1), lambda i: (i, 0)),
            pl.BlockSpec((D,), lambda i: (0,)),
            pl.BlockSpec((D, D), lambda i: (0, 0)),
        ],
        out_specs=pl.BlockSpec((_BLK, D), lambda i: (i, 0)),
        out_shape=jax.ShapeDtypeStruct((NPAD, D), jnp.float32),
    )(agg, dis, b1, W2)


def _tc3_body(agg_ref, dis_ref, b_ref, o_ref):
    o_ref[...] = jnp.maximum(
        agg_ref[...] * dis_ref[...] + b_ref[...][None, :], 0.0)


def _tc3(agg, dis, b2):
    return pl.pallas_call(
        _tc3_body,
        grid=(NPAD // _BLK,),
        in_specs=[
            pl.BlockSpec((_BLK, D), lambda i: (i, 0)),
            pl.BlockSpec((_BLK, 1), lambda i: (i, 0)),
            pl.BlockSpec((D,), lambda i: (0,)),
        ],
        out_specs=pl.BlockSpec((_BLK, D), lambda i: (i, 0)),
        out_shape=jax.ShapeDtypeStruct((N, D), jnp.float32),
    )(agg, dis, b2)


def kernel(x, edge_index, W1, b1, W2, b2):
    edges = edge_index.reshape(2 * E)
    zeros_row = jnp.zeros((NPAD,), jnp.float32)
    safe_src = jnp.zeros((K,), jnp.int32)
    safe_dst = jnp.full((K,), N, jnp.int32)

    deg_parts = _deg_kernel(edges, zeros_row)
    y1, dis = _tc1(deg_parts, x, W1)
    agg1 = _agg_kernel(y1, edges, safe_src, safe_dst)
    y2 = _tc2(agg1, dis, b1, W2)
    agg2 = _agg_kernel(y2, edges, safe_src, safe_dst)
    return _tc3(agg2, dis, b2)
